# Initial kernel scaffold; baseline (speedup 1.0000x reference)
import jax
import jax.numpy as jnp
from jax.experimental import pallas as pl


def kernel(x, W0, g1, b1, W1a, b1a, W1b, b1b, g2, b2, W2a, b2a, W2b, b2b, gF, bF, Wout, keys_pk, logit_scale):
    raise NotImplementedError("write your pallas kernel here")


# speedup vs baseline: 3.4041x; 3.4041x over previous
import jax
import jax.numpy as jnp
from jax import lax
from jax.experimental import pallas as pl
from jax.experimental.pallas import tpu as pltpu

DIM = 2048
HID = 2 * DIM
KDIM = 256
NUM_KEYS = 256
NC = 32
B = 2
S = 2048
TOK = B * S

_PREC = lax.Precision.DEFAULT


_TK_TB = 256
_FRONTIER_COUNTS = [32 // (i + 1) for i in range(32)]
_NFRONT = sum(_FRONTIER_COUNTS)
_NPAD = 128


def _topk32(s, tb, n):
    cols = lax.broadcasted_iota(jnp.int32, (tb, n), 1)
    vals = s
    out_v, out_i = [], []
    for _ in range(NC):
        m = jnp.max(vals, axis=1, keepdims=True)
        pos = jnp.min(jnp.where(vals == m, cols, n), axis=1, keepdims=True)
        out_v.append(m)
        out_i.append(pos)
        vals = jnp.where(cols == pos, jnp.float32(-jnp.inf), vals)
    return jnp.concatenate(out_v, axis=1), jnp.concatenate(out_i, axis=1)


def _pk_topk_body(s1_ref, s2_ref, oi_ref, os_ref):
    ts1, ti1 = _topk32(s1_ref[...], _TK_TB, NUM_KEYS)
    ts2, ti2 = _topk32(s2_ref[...], _TK_TB, NUM_KEYS)

    pieces_v, pieces_i = [], []
    for i, c in enumerate(_FRONTIER_COUNTS):
        pieces_v.append(ts1[:, i:i + 1] + ts2[:, :c])
        pieces_i.append(ti1[:, i:i + 1] * NUM_KEYS + ti2[:, :c])
    pieces_v.append(jnp.full((_TK_TB, _NPAD - _NFRONT), -jnp.inf, jnp.float32))
    pieces_i.append(jnp.zeros((_TK_TB, _NPAD - _NFRONT), jnp.int32))
    cv = jnp.concatenate(pieces_v, axis=1)
    ci = jnp.concatenate(pieces_i, axis=1)

    cols = lax.broadcasted_iota(jnp.int32, (_TK_TB, _NPAD), 1)
    out_v, out_i = [], []
    vals = cv
    for _ in range(NC):
        mx = jnp.max(vals, axis=1, keepdims=True)
        pos = jnp.min(jnp.where(vals == mx, cols, _NPAD), axis=1, keepdims=True)
        sel = cols == pos
        out_v.append(mx)
        out_i.append(jnp.sum(jnp.where(sel, ci, 0), axis=1, keepdims=True))
        vals = jnp.where(sel, jnp.float32(-jnp.inf), vals)
    ts = jnp.concatenate(out_v, axis=1)
    ti = jnp.concatenate(out_i, axis=1)

    mxx = jnp.max(ts, axis=1, keepdims=True)
    e = jnp.exp(ts - mxx)
    sm = e / jnp.sum(e, axis=1, keepdims=True)

    oi_ref[...] = ti
    os_ref[...] = sm


def _pk_topk(s1, s2):
    return pl.pallas_call(
        _pk_topk_body,
        grid=(TOK // _TK_TB,),
        in_specs=[
            pl.BlockSpec((_TK_TB, NUM_KEYS), lambda t: (t, 0)),
            pl.BlockSpec((_TK_TB, NUM_KEYS), lambda t: (t, 0)),
        ],
        out_specs=[
            pl.BlockSpec((_TK_TB, NC), lambda t: (t, 0)),
            pl.BlockSpec((_TK_TB, NC), lambda t: (t, 0)),
        ],
        out_shape=[
            jax.ShapeDtypeStruct((TOK, NC), jnp.int32),
            jax.ShapeDtypeStruct((TOK, NC), jnp.float32),
        ],
    )(s1, s2)


def _layer_norm(x, g, b, eps=1e-5):
    m = jnp.mean(x, axis=-1, keepdims=True)
    v = jnp.var(x, axis=-1, keepdims=True)
    return (x - m) / jnp.sqrt(v + eps) * g + b


def _l2norm(x, eps=1e-12):
    n = jnp.sqrt(jnp.sum(x * x, axis=-1, keepdims=True))
    return x / jnp.maximum(n, eps)


def kernel(x, W0, g1, b1, W1a, b1a, W1b, b1b, g2, b2, W2a, b2a, W2b, b2b,
           gF, bF, Wout, keys_pk, logit_scale):
    q = x @ W0.T

    def block(h, g, b, Wa, ba, Wb, bb):
        hn = _layer_norm(h, g, b)
        t = hn @ Wa.T + ba
        return h + (jax.nn.gelu(t, approximate=False) @ Wb.T + bb)

    q = block(q, g1, b1, W1a, b1a, W1b, b1b)
    q = block(q, g2, b2, W2a, b2a, W2b, b2b)
    q = _layer_norm(q, gF, bF)
    q = q @ Wout.T

    q1 = _l2norm(q[..., : KDIM // 2])
    q2 = _l2norm(q[..., KDIM // 2:])
    k1 = _l2norm(keys_pk[0])
    k2 = _l2norm(keys_pk[1])
    ls = jnp.minimum(jnp.exp(logit_scale), 100.0)
    scores_1 = jnp.einsum('bsd,kd->bsk', q1, k1) * ls
    scores_2 = jnp.einsum('bsd,kd->bsk', q2, k2) * ls

    ti, sm = _pk_topk(scores_1.reshape(TOK, NUM_KEYS),
                      scores_2.reshape(TOK, NUM_KEYS))
    return ti.reshape(B, S, NC), sm.reshape(B, S, NC)


# speedup vs baseline: 3.4102x; 1.0018x over previous
import jax
import jax.numpy as jnp
from jax import lax
from jax.experimental import pallas as pl

DIM = 2048
HID = 2 * DIM
KDIM = 256
NUM_KEYS = 256
NC = 32
B = 2
S = 2048
TOK = B * S

_PREC = lax.Precision.DEFAULT


_TK_TB = 256
_FRONTIER_COUNTS = [32 // (i + 1) for i in range(32)]
_NFRONT = sum(_FRONTIER_COUNTS)
_NPAD = 128


def _topk32(s, tb, n):
    cols = lax.broadcasted_iota(jnp.int32, (tb, n), 1)
    vals = s
    out_v, out_i = [], []
    for _ in range(NC):
        m = jnp.max(vals, axis=1, keepdims=True)
        pos = jnp.min(jnp.where(vals == m, cols, n), axis=1, keepdims=True)
        out_v.append(m)
        out_i.append(pos)
        vals = jnp.where(cols == pos, jnp.float32(-jnp.inf), vals)
    return jnp.concatenate(out_v, axis=1), jnp.concatenate(out_i, axis=1)


def _pk_topk_body(s1_ref, s2_ref, oi_ref, os_ref):
    ts1, ti1 = _topk32(s1_ref[...], _TK_TB, NUM_KEYS)
    ts2, ti2 = _topk32(s2_ref[...], _TK_TB, NUM_KEYS)

    pieces_v, pieces_i = [], []
    for i, c in enumerate(_FRONTIER_COUNTS):
        pieces_v.append(ts1[:, i:i + 1] + ts2[:, :c])
        pieces_i.append(ti1[:, i:i + 1] * NUM_KEYS + ti2[:, :c])
    pieces_v.append(jnp.full((_TK_TB, _NPAD - _NFRONT), -jnp.inf, jnp.float32))
    pieces_i.append(jnp.zeros((_TK_TB, _NPAD - _NFRONT), jnp.int32))
    cv = jnp.concatenate(pieces_v, axis=1)
    ci = jnp.concatenate(pieces_i, axis=1)

    cols = lax.broadcasted_iota(jnp.int32, (_TK_TB, _NPAD), 1)
    out_v, out_i = [], []
    vals = cv
    for _ in range(NC):
        mx = jnp.max(vals, axis=1, keepdims=True)
        pos = jnp.min(jnp.where(vals == mx, cols, _NPAD), axis=1, keepdims=True)
        sel = cols == pos
        out_v.append(mx)
        out_i.append(jnp.sum(jnp.where(sel, ci, 0), axis=1, keepdims=True))
        vals = jnp.where(sel, jnp.float32(-jnp.inf), vals)
    ts = jnp.concatenate(out_v, axis=1)
    ti = jnp.concatenate(out_i, axis=1)

    mxx = jnp.max(ts, axis=1, keepdims=True)
    e = jnp.exp(ts - mxx)
    sm = e / jnp.sum(e, axis=1, keepdims=True)

    oi_ref[...] = ti
    os_ref[...] = sm


def _pk_topk(s1, s2):
    return pl.pallas_call(
        _pk_topk_body,
        grid=(TOK // _TK_TB,),
        in_specs=[
            pl.BlockSpec((_TK_TB, NUM_KEYS), lambda t: (t, 0)),
            pl.BlockSpec((_TK_TB, NUM_KEYS), lambda t: (t, 0)),
        ],
        out_specs=[
            pl.BlockSpec((_TK_TB, NC), lambda t: (t, 0)),
            pl.BlockSpec((_TK_TB, NC), lambda t: (t, 0)),
        ],
        out_shape=[
            jax.ShapeDtypeStruct((TOK, NC), jnp.int32),
            jax.ShapeDtypeStruct((TOK, NC), jnp.float32),
        ],
    )(s1, s2)


def _layer_norm(x, g, b, eps=1e-5):
    m = jnp.mean(x, axis=-1, keepdims=True)
    v = jnp.var(x, axis=-1, keepdims=True)
    return (x - m) / jnp.sqrt(v + eps) * g + b


def _l2norm(x, eps=1e-12):
    n = jnp.sqrt(jnp.sum(x * x, axis=-1, keepdims=True))
    return x / jnp.maximum(n, eps)


def kernel(x, W0, g1, b1, W1a, b1a, W1b, b1b, g2, b2, W2a, b2a, W2b, b2b,
           gF, bF, Wout, keys_pk, logit_scale):
    q = x @ W0.T

    def block(h, g, b, Wa, ba, Wb, bb):
        hn = _layer_norm(h, g, b)
        t = hn @ Wa.T + ba
        return h + (jax.nn.gelu(t, approximate=False) @ Wb.T + bb)

    q = block(q, g1, b1, W1a, b1a, W1b, b1b)
    q = block(q, g2, b2, W2a, b2a, W2b, b2b)
    q = _layer_norm(q, gF, bF)
    q = q @ Wout.T

    q1 = _l2norm(q[..., : KDIM // 2])
    q2 = _l2norm(q[..., KDIM // 2:])
    k1 = _l2norm(keys_pk[0])
    k2 = _l2norm(keys_pk[1])
    ls = jnp.minimum(jnp.exp(logit_scale), 100.0)
    scores_1 = jnp.einsum('bsd,kd->bsk', q1, k1) * ls
    scores_2 = jnp.einsum('bsd,kd->bsk', q2, k2) * ls

    ti, sm = _pk_topk(scores_1.reshape(TOK, NUM_KEYS),
                      scores_2.reshape(TOK, NUM_KEYS))
    return ti.reshape(B, S, NC), sm.reshape(B, S, NC)
